# baseline (device time: 93495 ns/iter reference)
import jax
import jax.numpy as jnp
from jax import lax
from jax.experimental import pallas as pl
from jax.experimental.pallas import tpu as pltpu

N_DEV = 4


def kernel(x, w_mat):
    x = x.astype(jnp.bfloat16)
    w = w_mat.astype(jnp.bfloat16)

    m_total, k_per = x.shape
    k_total, n = w.shape
    m_per = m_total // N_DEV

    def body(x_ref, w_ref, out_ref, comm_ref, send_sems, recv_sems):
        my = lax.axis_index("i")

        barrier = pltpu.get_barrier_semaphore()
        for d in range(1, N_DEV):
            peer = lax.rem(my + d, N_DEV)
            pl.semaphore_signal(
                barrier, inc=1,
                device_id=(peer,), device_id_type=pl.DeviceIdType.MESH,
            )
        pl.semaphore_wait(barrier, N_DEV - 1)

        rdmas = []
        for d in range(1, N_DEV):
            peer = lax.rem(my + d, N_DEV)
            rdma = pltpu.make_async_remote_copy(
                src_ref=x_ref.at[pl.ds(peer * m_per, m_per), :],
                dst_ref=comm_ref.at[d - 1],
                send_sem=send_sems.at[d - 1],
                recv_sem=recv_sems.at[d - 1],
                device_id=(peer,),
                device_id_type=pl.DeviceIdType.MESH,
            )
            rdma.start()
            rdmas.append(rdma)

        x_own = x_ref[pl.ds(my * m_per, m_per), :]
        w_own = w_ref[pl.ds(my * k_per, k_per), :]
        out_ref[...] = jnp.dot(x_own, w_own, preferred_element_type=jnp.float32)

        for d in range(1, N_DEV):
            rdmas[d - 1].wait_recv()
            kb = lax.rem(my - d + N_DEV, N_DEV)
            w_blk = w_ref[pl.ds(kb * k_per, k_per), :]
            out_ref[...] += jnp.dot(
                comm_ref[d - 1], w_blk, preferred_element_type=jnp.float32
            )

        for r in rdmas:
            r.wait_send()

        y = out_ref[...]
        out_ref[...] = y * jax.nn.sigmoid(y)

    return pl.pallas_call(
        body,
        out_shape=jax.ShapeDtypeStruct((m_per, n), jnp.float32),
        in_specs=[
            pl.BlockSpec(memory_space=pltpu.VMEM),
            pl.BlockSpec(memory_space=pltpu.VMEM),
        ],
        out_specs=pl.BlockSpec(memory_space=pltpu.VMEM),
        scratch_shapes=[
            pltpu.VMEM((N_DEV - 1, m_per, k_per), jnp.bfloat16),
            pltpu.SemaphoreType.DMA((N_DEV - 1,)),
            pltpu.SemaphoreType.DMA((N_DEV - 1,)),
        ],
        compiler_params=pltpu.CompilerParams(collective_id=0),
    )(x, w)


# device time: 75954 ns/iter; 1.2309x vs baseline; 1.2309x over previous
import jax
import jax.numpy as jnp
from jax import lax
from jax.experimental import pallas as pl
from jax.experimental.pallas import tpu as pltpu

N_DEV = 4


def kernel(x, w_mat):
    m_total, k_per = x.shape
    k_total, n = w_mat.shape
    m_per = m_total // N_DEV

    def body(x_ref, w_hbm, out_ref, xb_ref, comm_ref, wstage_ref,
             send_sems, recv_sems, wsems):
        my = lax.axis_index("i")

        barrier = pltpu.get_barrier_semaphore()
        for d in range(1, N_DEV):
            peer = lax.rem(my + d, N_DEV)
            pl.semaphore_signal(
                barrier, inc=1,
                device_id=(peer,), device_id_type=pl.DeviceIdType.MESH,
            )
        pl.semaphore_wait(barrier, N_DEV - 1)

        def w_copy(kb, slot):
            return pltpu.make_async_copy(
                w_hbm.at[pl.ds(kb * k_per, k_per), :],
                wstage_ref.at[slot],
                wsems.at[slot],
            )

        wc = [w_copy(my, 0)]
        wc[0].start()

        rdmas = []
        for d in range(1, N_DEV):
            peer = lax.rem(my + d, N_DEV)
            xb_ref[d - 1] = x_ref[pl.ds(peer * m_per, m_per), :].astype(
                jnp.bfloat16
            )
            rdma = pltpu.make_async_remote_copy(
                src_ref=xb_ref.at[d - 1],
                dst_ref=comm_ref.at[d - 1],
                send_sem=send_sems.at[d - 1],
                recv_sem=recv_sems.at[d - 1],
                device_id=(peer,),
                device_id_type=pl.DeviceIdType.MESH,
            )
            rdma.start()
            rdmas.append(rdma)

        xb_ref[N_DEV - 1] = x_ref[pl.ds(my * m_per, m_per), :].astype(
            jnp.bfloat16
        )

        wc.append(w_copy(lax.rem(my + N_DEV - 1, N_DEV), 1))
        wc[1].start()

        wc[0].wait()
        out_ref[...] = jnp.dot(
            xb_ref[N_DEV - 1],
            wstage_ref[0].astype(jnp.bfloat16),
            preferred_element_type=jnp.float32,
        )

        for d in range(1, N_DEV):
            slot = d % 2
            if d + 1 < N_DEV:
                kb_next = lax.rem(my - (d + 1) + N_DEV, N_DEV)
                nxt = w_copy(kb_next, (d + 1) % 2)
                nxt.start()
                wc.append(nxt)
            rdmas[d - 1].wait_recv()
            wc[d].wait()
            out_ref[...] += jnp.dot(
                comm_ref[d - 1],
                wstage_ref[slot].astype(jnp.bfloat16),
                preferred_element_type=jnp.float32,
            )

        for r in rdmas:
            r.wait_send()

        y = out_ref[...]
        out_ref[...] = y * jax.nn.sigmoid(y)

    return pl.pallas_call(
        body,
        out_shape=jax.ShapeDtypeStruct((m_per, n), jnp.float32),
        in_specs=[
            pl.BlockSpec(memory_space=pltpu.VMEM),
            pl.BlockSpec(memory_space=pl.ANY),
        ],
        out_specs=pl.BlockSpec(memory_space=pltpu.VMEM),
        scratch_shapes=[
            pltpu.VMEM((N_DEV, m_per, k_per), jnp.bfloat16),
            pltpu.VMEM((N_DEV - 1, m_per, k_per), jnp.bfloat16),
            pltpu.VMEM((2, k_per, n), jnp.float32),
            pltpu.SemaphoreType.DMA((N_DEV - 1,)),
            pltpu.SemaphoreType.DMA((N_DEV - 1,)),
            pltpu.SemaphoreType.DMA((2,)),
        ],
        compiler_params=pltpu.CompilerParams(
            collective_id=0,
            vmem_limit_bytes=100 * 1024 * 1024,
        ),
    )(x, w_mat)


# device time: 72646 ns/iter; 1.2870x vs baseline; 1.0455x over previous
import jax
import jax.numpy as jnp
from jax import lax
from jax.experimental import pallas as pl
from jax.experimental.pallas import tpu as pltpu

N_DEV = 4


def kernel(x, w_mat):
    m_total, k_per = x.shape
    k_total, n = w_mat.shape
    m_per = m_total // N_DEV

    def body(x_hbm, w_hbm, out_ref, xstage_ref, xb_ref, comm_ref,
             wstage_ref, xsems, send_sems, recv_sems, wsems):
        my = lax.axis_index("i")

        def x_copy(row, slot):
            return pltpu.make_async_copy(
                x_hbm.at[pl.ds(row * m_per, m_per), :],
                xstage_ref.at[slot],
                xsems.at[slot],
            )

        xcs = []
        for d in range(1, N_DEV):
            c = x_copy(lax.rem(my + d, N_DEV), d - 1)
            c.start()
            xcs.append(c)
        xc_own = x_copy(my, N_DEV - 1)
        xc_own.start()

        def w_copy(kb, slot):
            return pltpu.make_async_copy(
                w_hbm.at[pl.ds(kb * k_per, k_per), :],
                wstage_ref.at[slot],
                wsems.at[slot],
            )

        wc = [w_copy(my, 0)]
        wc[0].start()

        barrier = pltpu.get_barrier_semaphore()
        for d in range(1, N_DEV):
            peer = lax.rem(my + d, N_DEV)
            pl.semaphore_signal(
                barrier, inc=1,
                device_id=(peer,), device_id_type=pl.DeviceIdType.MESH,
            )
        pl.semaphore_wait(barrier, N_DEV - 1)

        rdmas = []
        for d in range(1, N_DEV):
            peer = lax.rem(my + d, N_DEV)
            xcs[d - 1].wait()
            xb_ref[d - 1] = xstage_ref[d - 1].astype(jnp.bfloat16)
            rdma = pltpu.make_async_remote_copy(
                src_ref=xb_ref.at[d - 1],
                dst_ref=comm_ref.at[d - 1],
                send_sem=send_sems.at[d - 1],
                recv_sem=recv_sems.at[d - 1],
                device_id=(peer,),
                device_id_type=pl.DeviceIdType.MESH,
            )
            rdma.start()
            rdmas.append(rdma)

        wc.append(w_copy(lax.rem(my + N_DEV - 1, N_DEV), 1))
        wc[1].start()

        xc_own.wait()
        wc[0].wait()
        out_ref[...] = jnp.dot(
            xstage_ref[N_DEV - 1].astype(jnp.bfloat16),
            wstage_ref[0].astype(jnp.bfloat16),
            preferred_element_type=jnp.float32,
        )

        for d in range(1, N_DEV):
            slot = d % 2
            if d + 1 < N_DEV:
                kb_next = lax.rem(my - (d + 1) + N_DEV, N_DEV)
                nxt = w_copy(kb_next, (d + 1) % 2)
                nxt.start()
                wc.append(nxt)
            rdmas[d - 1].wait_recv()
            wc[d].wait()
            out_ref[...] += jnp.dot(
                comm_ref[d - 1],
                wstage_ref[slot].astype(jnp.bfloat16),
                preferred_element_type=jnp.float32,
            )

        for r in rdmas:
            r.wait_send()

        y = out_ref[...]
        out_ref[...] = y * jax.nn.sigmoid(y)

    return pl.pallas_call(
        body,
        out_shape=jax.ShapeDtypeStruct((m_per, n), jnp.float32),
        in_specs=[
            pl.BlockSpec(memory_space=pl.ANY),
            pl.BlockSpec(memory_space=pl.ANY),
        ],
        out_specs=pl.BlockSpec(memory_space=pltpu.VMEM),
        scratch_shapes=[
            pltpu.VMEM((N_DEV, m_per, k_per), jnp.float32),
            pltpu.VMEM((N_DEV - 1, m_per, k_per), jnp.bfloat16),
            pltpu.VMEM((N_DEV - 1, m_per, k_per), jnp.bfloat16),
            pltpu.VMEM((2, k_per, n), jnp.float32),
            pltpu.SemaphoreType.DMA((N_DEV,)),
            pltpu.SemaphoreType.DMA((N_DEV - 1,)),
            pltpu.SemaphoreType.DMA((N_DEV - 1,)),
            pltpu.SemaphoreType.DMA((2,)),
        ],
        compiler_params=pltpu.CompilerParams(
            collective_id=0,
            vmem_limit_bytes=100 * 1024 * 1024,
        ),
    )(x, w_mat)
